# baseline (speedup 1.0000x reference)
import jax
import jax.numpy as jnp
from jax import lax
from jax.experimental import pallas as pl
from jax.experimental.pallas import tpu as pltpu
from jax.experimental.pallas import tpu_sc as plsc

_N = 10000
_E = 320000
_H = 128
_BN_EPS = 1e-5

_NC = 2
_NS = 16
_NW = _NC * _NS
_CH = 128
_CPW0 = 158
_E0 = _NS * _CPW0 * _CH
_RPT = 632
_NP = _NS * _RPT


def _agg_body(h_hbm, src_hbm, dst_hbm, out_hbm,
              sring, dring, buf, acc,
              gsem0, gsem1, ssem0, ssem1, dsem0, dsem1):
    c = lax.axis_index("c")
    s = lax.axis_index("s")

    @pl.when(c == 0)
    def _():
        _agg_core0(h_hbm, src_hbm, dst_hbm, out_hbm, sring, dring, buf, acc,
                   gsem0, gsem1, ssem0, ssem1, dsem0, dsem1, s, _CPW0)


def _agg_core0(h_hbm, src_hbm, dst_hbm, out_hbm, sring, dring, buf, acc,
               gsem0, gsem1, ssem0, ssem1, dsem0, dsem1, s, cpw):
    wid = s

    z16 = jnp.zeros((16,), jnp.float32)

    @pl.loop(0, _CH)
    def _(r):
        for k in range(8):
            buf[0, r, pl.ds(k * 16, 16)] = z16
    for i in range(4):
        pltpu.sync_copy(buf.at[0], acc.at[pl.ds(s * _RPT + i * _CH, _CH)])
    pltpu.sync_copy(buf.at[0, pl.ds(0, _RPT - 4 * _CH)],
                    acc.at[pl.ds(s * _RPT + 4 * _CH, _RPT - 4 * _CH)])

    pltpu.async_copy(src_hbm.at[wid, 0], sring.at[0], ssem0)
    pltpu.async_copy(src_hbm.at[wid, 1], sring.at[1], ssem1)
    pltpu.async_copy(dst_hbm.at[wid, 0], dring.at[0], dsem0)
    pltpu.async_copy(dst_hbm.at[wid, 1], dring.at[1], dsem1)
    plsc.subcore_barrier()
    pltpu.make_async_copy(src_hbm.at[wid, 0], sring.at[0], ssem0).wait()
    pltpu.async_copy(h_hbm.at[sring.at[0]], buf.at[0], gsem0)
    pltpu.make_async_copy(src_hbm.at[wid, 1], sring.at[1], ssem1).wait()
    pltpu.async_copy(h_hbm.at[sring.at[1]], buf.at[1], gsem1)

    @pl.loop(0, cpw, step=2)
    def _(j):
        pltpu.make_async_copy(h_hbm.at[sring.at[0]], buf.at[0], gsem0).wait()

        @pl.when(j + 2 < cpw)
        def _():
            pltpu.async_copy(src_hbm.at[wid, j + 2], sring.at[0], ssem0)

        pltpu.make_async_copy(dst_hbm.at[wid, 0], dring.at[0], dsem0).wait()
        pltpu.sync_copy(buf.at[0], acc.at[dring.at[0]], add=True)

        @pl.when(j + 2 < cpw)
        def _():
            pltpu.async_copy(dst_hbm.at[wid, j + 2], dring.at[0], dsem0)
            pltpu.make_async_copy(src_hbm.at[wid, 0], sring.at[0], ssem0).wait()
            pltpu.async_copy(h_hbm.at[sring.at[0]], buf.at[0], gsem0)

        pltpu.make_async_copy(h_hbm.at[sring.at[1]], buf.at[1], gsem1).wait()

        @pl.when(j + 3 < cpw)
        def _():
            pltpu.async_copy(src_hbm.at[wid, j + 3], sring.at[1], ssem1)

        pltpu.make_async_copy(dst_hbm.at[wid, 0], dring.at[1], dsem1).wait()
        pltpu.sync_copy(buf.at[1], acc.at[dring.at[1]], add=True)

        @pl.when(j + 3 < cpw)
        def _():
            pltpu.async_copy(dst_hbm.at[wid, j + 3], dring.at[1], dsem1)
            pltpu.make_async_copy(src_hbm.at[wid, 0], sring.at[1], ssem1).wait()
            pltpu.async_copy(h_hbm.at[sring.at[1]], buf.at[1], gsem1)

    plsc.subcore_barrier()
    pltpu.sync_copy(acc.at[pl.ds(s * _RPT, _RPT)],
                    out_hbm.at[0, pl.ds(s * _RPT, _RPT)])


def _aggregate(h_pad, src_g, dst_g):
    mesh = plsc.VectorSubcoreMesh(core_axis_name="c", subcore_axis_name="s")
    return pl.kernel(
        _agg_body,
        out_type=jax.ShapeDtypeStruct((1, _NP, _H), jnp.float32),
        mesh=mesh,
        scratch_types=[
            pltpu.VMEM((2, _CH), jnp.int32),
            pltpu.VMEM((2, _CH), jnp.int32),
            pltpu.VMEM((2, _CH, _H), jnp.float32),
            pltpu.VMEM_SHARED((_NP, _H), jnp.float32),
            pltpu.SemaphoreType.DMA,
            pltpu.SemaphoreType.DMA,
            pltpu.SemaphoreType.DMA,
            pltpu.SemaphoreType.DMA,
            pltpu.SemaphoreType.DMA,
            pltpu.SemaphoreType.DMA,
        ],
    )(h_pad, src_g, dst_g)


def _mlp_body(x_ref, agg_ref, W1_ref, b1_ref, g_ref, beta_ref, W2_ref, b2_ref,
              eps_ref, out_ref):
    x = x_ref[: _N, :]
    agg = agg_ref[0, : _N, :]
    h = (1.0 + eps_ref[0]) * x + agg
    t = jnp.dot(h, W1_ref[...], preferred_element_type=jnp.float32) + b1_ref[...]
    mu = jnp.mean(t, axis=0, keepdims=True)
    d = t - mu
    var = jnp.mean(d * d, axis=0, keepdims=True)
    r = d * jax.lax.rsqrt(var + _BN_EPS) * g_ref[...] + beta_ref[...]
    r = jnp.maximum(r, 0.0)
    o = jnp.dot(r, W2_ref[...], preferred_element_type=jnp.float32) + b2_ref[...]
    out_ref[: _N, :] = o
    out_ref[_N:, :] = jnp.zeros((_NP - _N, _H), jnp.float32)


def _mlp(x_pad, agg, eps, W1, b1, g, beta, W2, b2):
    eps_s = jnp.reshape(eps, (1,))
    return pl.pallas_call(
        _mlp_body,
        out_shape=jax.ShapeDtypeStruct((_NP, _H), jnp.float32),
        in_specs=[pl.BlockSpec(memory_space=pltpu.VMEM)] * 8
        + [pl.BlockSpec(memory_space=pltpu.SMEM)],
        out_specs=pl.BlockSpec(memory_space=pltpu.VMEM),
    )(x_pad, agg, W1, jnp.reshape(b1, (1, -1)), jnp.reshape(g, (1, -1)),
      jnp.reshape(beta, (1, -1)), W2, jnp.reshape(b2, (1, -1)), eps_s)


def _mlp_final_body(x_ref, agg_ref, W1_ref, b1_ref, g_ref, beta_ref, W2_ref,
                    b2_ref, lw_ref, lb_ref, eps_ref, out_ref):
    x = x_ref[: _N, :]
    agg = agg_ref[0, : _N, :]
    h = (1.0 + eps_ref[0]) * x + agg
    t = jnp.dot(h, W1_ref[...], preferred_element_type=jnp.float32) + b1_ref[...]
    mu = jnp.mean(t, axis=0, keepdims=True)
    d = t - mu
    var = jnp.mean(d * d, axis=0, keepdims=True)
    r = d * jax.lax.rsqrt(var + _BN_EPS) * g_ref[...] + beta_ref[...]
    r = jnp.maximum(r, 0.0)
    o = jnp.dot(r, W2_ref[...], preferred_element_type=jnp.float32) + b2_ref[...]
    out_ref[...] = (
        jnp.dot(o, lw_ref[...], preferred_element_type=jnp.float32) + lb_ref[...]
    )


def _mlp_final(x_pad, agg, eps, W1, b1, g, beta, W2, b2, lin_W, lin_b):
    eps_s = jnp.reshape(eps, (1,))
    return pl.pallas_call(
        _mlp_final_body,
        out_shape=jax.ShapeDtypeStruct((_N, _H), jnp.float32),
        in_specs=[pl.BlockSpec(memory_space=pltpu.VMEM)] * 10
        + [pl.BlockSpec(memory_space=pltpu.SMEM)],
        out_specs=pl.BlockSpec(memory_space=pltpu.VMEM),
    )(x_pad, agg, W1, jnp.reshape(b1, (1, -1)), jnp.reshape(g, (1, -1)),
      jnp.reshape(beta, (1, -1)), W2, jnp.reshape(b2, (1, -1)), lin_W,
      jnp.reshape(lin_b, (1, -1)), eps_s)


def kernel(x, edge_index, W1_0, b1_0, g_0, beta_0, W2_0, b2_0, eps_0,
           W1_1, b1_1, g_1, beta_1, W2_1, b2_1, eps_1,
           W1_2, b1_2, g_2, beta_2, W2_2, b2_2, eps_2, lin_W, lin_b):
    def shard(v):
        a = jnp.concatenate([v, jnp.full((_E0 - _E,), _N, jnp.int32)])
        return a.reshape(_NS, _CPW0, _CH)

    src_g = shard(edge_index[0])
    dst_g = shard(edge_index[1])
    h = jnp.pad(x, ((0, _NP - _N), (0, 0)))

    layers = (
        (eps_0, W1_0, b1_0, g_0, beta_0, W2_0, b2_0),
        (eps_1, W1_1, b1_1, g_1, beta_1, W2_1, b2_1),
    )
    for eps, W1, b1, g, beta, W2, b2 in layers:
        agg = _aggregate(h, src_g, dst_g)
        h = _mlp(h, agg, eps, W1, b1, g, beta, W2, b2)

    agg = _aggregate(h, src_g, dst_g)
    return _mlp_final(h, agg, eps_2, W1_2, b1_2, g_2, beta_2, W2_2, b2_2,
                      lin_W, lin_b)


# speedup vs baseline: 1.4123x; 1.4123x over previous
import jax
import jax.numpy as jnp
from jax import lax
from jax.experimental import pallas as pl
from jax.experimental.pallas import tpu as pltpu
from jax.experimental.pallas import tpu_sc as plsc

_N = 10000
_E = 320000
_H = 128
_BN_EPS = 1e-5

_NC = 2
_NS = 16
_NW = _NC * _NS
_CH = 128
_CPW0 = 120
_CPW1 = 38
_CPWM = _CPW0
_E0 = _NS * _CPW0 * _CH
_E1 = _NS * _CPW1 * _CH
_RPT = 632
_NP = _NS * _RPT


def _agg_body(h_hbm, src_hbm, dst_hbm, out_hbm,
              sring, dring, buf, acc,
              gsem0, gsem1, ssem0, ssem1, dsem0, dsem1):
    c = lax.axis_index("c")
    s = lax.axis_index("s")
    wid = c * _NS + s
    cpw = jnp.where(c == 0, _CPW0, _CPW1)

    z16 = jnp.zeros((16,), jnp.float32)

    @pl.loop(0, _CH)
    def _(r):
        for k in range(8):
            buf[0, r, pl.ds(k * 16, 16)] = z16
    for i in range(4):
        pltpu.sync_copy(buf.at[0], acc.at[pl.ds(s * _RPT + i * _CH, _CH)])
    pltpu.sync_copy(buf.at[0, pl.ds(0, _RPT - 4 * _CH)],
                    acc.at[pl.ds(s * _RPT + 4 * _CH, _RPT - 4 * _CH)])

    pltpu.async_copy(src_hbm.at[wid, 0], sring.at[0], ssem0)
    pltpu.async_copy(src_hbm.at[wid, 1], sring.at[1], ssem1)
    pltpu.async_copy(dst_hbm.at[wid, 0], dring.at[0], dsem0)
    pltpu.async_copy(dst_hbm.at[wid, 1], dring.at[1], dsem1)
    plsc.subcore_barrier()
    pltpu.make_async_copy(src_hbm.at[wid, 0], sring.at[0], ssem0).wait()
    pltpu.async_copy(h_hbm.at[sring.at[0]], buf.at[0], gsem0)
    pltpu.make_async_copy(src_hbm.at[wid, 1], sring.at[1], ssem1).wait()
    pltpu.async_copy(h_hbm.at[sring.at[1]], buf.at[1], gsem1)

    @pl.loop(0, cpw, step=2)
    def _(j):
        pltpu.make_async_copy(h_hbm.at[sring.at[0]], buf.at[0], gsem0).wait()

        @pl.when(j + 2 < cpw)
        def _():
            pltpu.async_copy(src_hbm.at[wid, j + 2], sring.at[0], ssem0)

        pltpu.make_async_copy(dst_hbm.at[wid, 0], dring.at[0], dsem0).wait()
        pltpu.sync_copy(buf.at[0], acc.at[dring.at[0]], add=True)

        @pl.when(j + 2 < cpw)
        def _():
            pltpu.async_copy(dst_hbm.at[wid, j + 2], dring.at[0], dsem0)
            pltpu.make_async_copy(src_hbm.at[wid, 0], sring.at[0], ssem0).wait()
            pltpu.async_copy(h_hbm.at[sring.at[0]], buf.at[0], gsem0)

        pltpu.make_async_copy(h_hbm.at[sring.at[1]], buf.at[1], gsem1).wait()

        @pl.when(j + 3 < cpw)
        def _():
            pltpu.async_copy(src_hbm.at[wid, j + 3], sring.at[1], ssem1)

        pltpu.make_async_copy(dst_hbm.at[wid, 0], dring.at[1], dsem1).wait()
        pltpu.sync_copy(buf.at[1], acc.at[dring.at[1]], add=True)

        @pl.when(j + 3 < cpw)
        def _():
            pltpu.async_copy(dst_hbm.at[wid, j + 3], dring.at[1], dsem1)
            pltpu.make_async_copy(src_hbm.at[wid, 0], sring.at[1], ssem1).wait()
            pltpu.async_copy(h_hbm.at[sring.at[1]], buf.at[1], gsem1)

    plsc.subcore_barrier()
    pltpu.sync_copy(acc.at[pl.ds(s * _RPT, _RPT)],
                    out_hbm.at[c, pl.ds(s * _RPT, _RPT)])


def _aggregate(h_pad, src_g, dst_g):
    mesh = plsc.VectorSubcoreMesh(core_axis_name="c", subcore_axis_name="s")
    return pl.kernel(
        _agg_body,
        out_type=jax.ShapeDtypeStruct((_NC, _NP, _H), jnp.float32),
        mesh=mesh,
        scratch_types=[
            pltpu.VMEM((2, _CH), jnp.int32),
            pltpu.VMEM((2, _CH), jnp.int32),
            pltpu.VMEM((2, _CH, _H), jnp.float32),
            pltpu.VMEM_SHARED((_NP, _H), jnp.float32),
            pltpu.SemaphoreType.DMA,
            pltpu.SemaphoreType.DMA,
            pltpu.SemaphoreType.DMA,
            pltpu.SemaphoreType.DMA,
            pltpu.SemaphoreType.DMA,
            pltpu.SemaphoreType.DMA,
        ],
    )(h_pad, src_g, dst_g)


def _mlp_body(x_ref, agg_ref, W1_ref, b1_ref, g_ref, beta_ref, W2_ref, b2_ref,
              eps_ref, out_ref):
    x = x_ref[: _N, :]
    agg = agg_ref[0, : _N, :] + agg_ref[1, : _N, :]
    h = (1.0 + eps_ref[0]) * x + agg
    t = jnp.dot(h, W1_ref[...], preferred_element_type=jnp.float32) + b1_ref[...]
    mu = jnp.mean(t, axis=0, keepdims=True)
    d = t - mu
    var = jnp.mean(d * d, axis=0, keepdims=True)
    r = d * jax.lax.rsqrt(var + _BN_EPS) * g_ref[...] + beta_ref[...]
    r = jnp.maximum(r, 0.0)
    o = jnp.dot(r, W2_ref[...], preferred_element_type=jnp.float32) + b2_ref[...]
    out_ref[: _N, :] = o
    out_ref[_N:, :] = jnp.zeros((_NP - _N, _H), jnp.float32)


def _mlp(x_pad, agg, eps, W1, b1, g, beta, W2, b2):
    eps_s = jnp.reshape(eps, (1,))
    return pl.pallas_call(
        _mlp_body,
        out_shape=jax.ShapeDtypeStruct((_NP, _H), jnp.float32),
        in_specs=[pl.BlockSpec(memory_space=pltpu.VMEM)] * 8
        + [pl.BlockSpec(memory_space=pltpu.SMEM)],
        out_specs=pl.BlockSpec(memory_space=pltpu.VMEM),
    )(x_pad, agg, W1, jnp.reshape(b1, (1, -1)), jnp.reshape(g, (1, -1)),
      jnp.reshape(beta, (1, -1)), W2, jnp.reshape(b2, (1, -1)), eps_s)


def _mlp_final_body(x_ref, agg_ref, W1_ref, b1_ref, g_ref, beta_ref, W2_ref,
                    b2_ref, lw_ref, lb_ref, eps_ref, out_ref):
    x = x_ref[: _N, :]
    agg = agg_ref[0, : _N, :] + agg_ref[1, : _N, :]
    h = (1.0 + eps_ref[0]) * x + agg
    t = jnp.dot(h, W1_ref[...], preferred_element_type=jnp.float32) + b1_ref[...]
    mu = jnp.mean(t, axis=0, keepdims=True)
    d = t - mu
    var = jnp.mean(d * d, axis=0, keepdims=True)
    r = d * jax.lax.rsqrt(var + _BN_EPS) * g_ref[...] + beta_ref[...]
    r = jnp.maximum(r, 0.0)
    o = jnp.dot(r, W2_ref[...], preferred_element_type=jnp.float32) + b2_ref[...]
    out_ref[...] = (
        jnp.dot(o, lw_ref[...], preferred_element_type=jnp.float32) + lb_ref[...]
    )


def _mlp_final(x_pad, agg, eps, W1, b1, g, beta, W2, b2, lin_W, lin_b):
    eps_s = jnp.reshape(eps, (1,))
    return pl.pallas_call(
        _mlp_final_body,
        out_shape=jax.ShapeDtypeStruct((_N, _H), jnp.float32),
        in_specs=[pl.BlockSpec(memory_space=pltpu.VMEM)] * 10
        + [pl.BlockSpec(memory_space=pltpu.SMEM)],
        out_specs=pl.BlockSpec(memory_space=pltpu.VMEM),
    )(x_pad, agg, W1, jnp.reshape(b1, (1, -1)), jnp.reshape(g, (1, -1)),
      jnp.reshape(beta, (1, -1)), W2, jnp.reshape(b2, (1, -1)), lin_W,
      jnp.reshape(lin_b, (1, -1)), eps_s)


def kernel(x, edge_index, W1_0, b1_0, g_0, beta_0, W2_0, b2_0, eps_0,
           W1_1, b1_1, g_1, beta_1, W2_1, b2_1, eps_1,
           W1_2, b1_2, g_2, beta_2, W2_2, b2_2, eps_2, lin_W, lin_b):
    def shard(v):
        a = v[:_E0]
        b = jnp.concatenate(
            [v[_E0:], jnp.full((_E0 + _E1 - _E,), _N, jnp.int32)])
        a = a.reshape(_NS, _CPW0, _CH)
        b = b.reshape(_NS, _CPW1, _CH)
        b = jnp.pad(b, ((0, 0), (0, _CPWM - _CPW1), (0, 0)))
        return jnp.concatenate([a, b], axis=0)

    src_g = shard(edge_index[0])
    dst_g = shard(edge_index[1])
    h = jnp.pad(x, ((0, _NP - _N), (0, 0)))

    layers = (
        (eps_0, W1_0, b1_0, g_0, beta_0, W2_0, b2_0),
        (eps_1, W1_1, b1_1, g_1, beta_1, W2_1, b2_1),
    )
    for eps, W1, b1, g, beta, W2, b2 in layers:
        agg = _aggregate(h, src_g, dst_g)
        h = _mlp(h, agg, eps, W1, b1, g, beta, W2, b2)

    agg = _aggregate(h, src_g, dst_g)
    return _mlp_final(h, agg, eps_2, W1_2, b1_2, g_2, beta_2, W2_2, b2_2,
                      lin_W, lin_b)
